# baseline (speedup 1.0000x reference)
import dataclasses
import functools

import jax
import jax.numpy as jnp
from jax import lax
from jax.experimental import pallas as pl
from jax.experimental.pallas import tpu as pltpu
from jax.experimental.pallas import tpu_sc as plsc

N_NODES = 10000
N_EDGES = 320000
D = 128

NC = 2
NS = 16
L = 16

CHUNK = 128
CPW0 = 80
CPW1 = 80
SPC = 40
MAX_STAGES = max(CPW0, CPW1) // SPC
N_CHUNKS = NS * (CPW0 + CPW1)
E_PAD = N_CHUNKS * CHUNK
RBLK = 80
N_RBLK = N_NODES // RBLK
RB_T = (N_RBLK + NS - 1) // NS


def _matmul_body(x_ref, w_ref, o_ref):
    o_ref[...] = jnp.dot(x_ref[...], w_ref[...],
                         preferred_element_type=jnp.float32)


def _combine_body(p_ref, b_ref, o_ref):
    o_ref[...] = p_ref[0] + p_ref[1] + b_ref[...]


def _sc_spmm(h, row2, col, val):
    mesh = plsc.VectorSubcoreMesh(core_axis_name="c", subcore_axis_name="s")
    cp = pltpu.CompilerParams()
    if "needs_layout_passes" in pltpu.CompilerParams.__dataclass_fields__:
        cp = dataclasses.replace(cp, needs_layout_passes=False)

    @functools.partial(
        pl.kernel,
        compiler_params=cp,
        out_type=jax.ShapeDtypeStruct((NC, N_NODES, D), jnp.float32),
        mesh=mesh,
        scratch_types=[
            pltpu.VMEM((SPC * CHUNK,), jnp.int32),
            pltpu.VMEM((SPC, CHUNK), jnp.int32),
            pltpu.VMEM((SPC * CHUNK,), jnp.float32),
            pltpu.VMEM((CHUNK, D), jnp.float32),
            pltpu.VMEM((CHUNK, D), jnp.float32),
            pltpu.VMEM_SHARED((N_NODES, D), jnp.float32),
            pltpu.SemaphoreType.DMA,
            pltpu.SemaphoreType.DMA,
        ],
    )
    def spmm_kernel(h_hbm, row_hbm, col_hbm, val_hbm, out_hbm,
                    col_v, row_v, val_v, rows_a, rows_b, acc_sh,
                    sem_a, sem_b):
        cid = lax.axis_index("c")
        sid = lax.axis_index("s")

        @pl.loop(0, RBLK)
        def _(e):
            for k in range(D // L):
                rows_a[e, pl.ds(k * L, L)] = jnp.zeros((L,), jnp.float32)

        @pl.loop(0, RB_T)
        def _(t):
            blk = sid + t * NS

            @pl.when(blk < N_RBLK)
            def _():
                pltpu.sync_copy(rows_a.at[pl.ds(0, RBLK)],
                                acc_sh.at[pl.ds(blk * RBLK, RBLK)])

        plsc.subcore_barrier()

        def start_gather(j, buf, sem):
            pltpu.async_copy(
                h_hbm.at[col_v.at[pl.ds(j * CHUNK, CHUNK)]], buf, sem)

        def wait_gather(j, buf, sem):
            pltpu.make_async_copy(
                h_hbm.at[col_v.at[pl.ds(j * CHUNK, CHUNK)]], buf, sem).wait()

        def scale(buf, j):
            @pl.loop(0, CHUNK // L)
            def _(g):
                base_e = j * CHUNK + g * L
                for e in range(L):
                    bcast = plsc.load_gather(
                        val_v, [jnp.full((L,), base_e + e, jnp.int32)])
                    r = g * L + e
                    for k in range(D // L):
                        sl = pl.ds(k * L, L)
                        buf[r, sl] = buf[r, sl] * bcast

        def scatter(buf, j):
            pltpu.sync_copy(buf, acc_sh.at[row_v.at[j]], add=True)

        cpw = jnp.where(cid == 0, CPW0, CPW1)
        cbase = cid * NS * CPW0 + sid * cpw

        for s in range(MAX_STAGES):

            @pl.when(s * SPC < cpw)
            def _():
                stage_c = cbase + s * SPC
                ebase = stage_c * CHUNK
                pltpu.sync_copy(col_hbm.at[pl.ds(ebase, SPC * CHUNK)], col_v)
                pltpu.sync_copy(val_hbm.at[pl.ds(ebase, SPC * CHUNK)], val_v)
                pltpu.sync_copy(row_hbm.at[pl.ds(stage_c, SPC)], row_v)

                @pl.loop(0, SPC)
                def _(j):
                    start_gather(j, rows_a, sem_a)
                    wait_gather(j, rows_a, sem_a)
                    scale(rows_a, j)
                    scatter(rows_a, j)

        plsc.subcore_barrier()

        @pl.loop(0, RB_T)
        def _(t):
            blk = sid + t * NS

            @pl.when(blk < N_RBLK)
            def _():
                pltpu.sync_copy(
                    acc_sh.at[pl.ds(blk * RBLK, RBLK)],
                    out_hbm.at[cid, pl.ds(blk * RBLK, RBLK)])

    return spmm_kernel(h, row2, col, val)


def kernel(x, edge_index, adj_values, W, b):
    row = edge_index[0].astype(jnp.int32)
    col = edge_index[1].astype(jnp.int32)
    val = adj_values.astype(jnp.float32)

    pad = E_PAD - N_EDGES
    row2 = jnp.pad(row, (0, pad)).reshape(N_CHUNKS, CHUNK)
    col = jnp.pad(col, (0, pad))
    val = jnp.pad(val, (0, pad))

    h = pl.pallas_call(
        _matmul_body,
        grid=(10,),
        in_specs=[
            pl.BlockSpec((N_NODES // 10, D), lambda i: (i, 0)),
            pl.BlockSpec((D, D), lambda i: (0, 0)),
        ],
        out_specs=pl.BlockSpec((N_NODES // 10, D), lambda i: (i, 0)),
        out_shape=jax.ShapeDtypeStruct((N_NODES, D), jnp.float32),
    )(x, W)

    partials = _sc_spmm(h, row2, col, val)

    b2 = b.reshape(1, D).astype(jnp.float32)
    out = pl.pallas_call(
        _combine_body,
        grid=(10,),
        in_specs=[
            pl.BlockSpec((NC, N_NODES // 10, D), lambda i: (0, i, 0)),
            pl.BlockSpec((1, D), lambda i: (0, 0)),
        ],
        out_specs=pl.BlockSpec((N_NODES // 10, D), lambda i: (i, 0)),
        out_shape=jax.ShapeDtypeStruct((N_NODES, D), jnp.float32),
    )(partials, b2)
    return out


# speedup vs baseline: 1.1349x; 1.1349x over previous
import dataclasses
import functools

import jax
import jax.numpy as jnp
from jax import lax
from jax.experimental import pallas as pl
from jax.experimental.pallas import tpu as pltpu
from jax.experimental.pallas import tpu_sc as plsc

N_NODES = 10000
N_EDGES = 320000
D = 128

NC = 2
NS = 16
L = 16

CHUNK = 128
CPW = 80
N_CHUNKS = NC * NS * CPW
E_PAD = N_CHUNKS * CHUNK
RBLK = 80
N_RBLK = N_NODES // RBLK
RB_T = (N_RBLK + NS - 1) // NS


def _matmul_body(x_ref, w_ref, o_ref):
    o_ref[...] = jnp.dot(x_ref[...], w_ref[...],
                         preferred_element_type=jnp.float32)


def _combine_body(p_ref, b_ref, o_ref):
    o_ref[...] = p_ref[0] + p_ref[1] + b_ref[...]


def _sc_spmm(h, row2, col, val):
    mesh = plsc.VectorSubcoreMesh(core_axis_name="c", subcore_axis_name="s")
    cp = pltpu.CompilerParams()
    if "needs_layout_passes" in pltpu.CompilerParams.__dataclass_fields__:
        cp = dataclasses.replace(cp, needs_layout_passes=False)

    @functools.partial(
        pl.kernel,
        compiler_params=cp,
        out_type=jax.ShapeDtypeStruct((NC, N_NODES, D), jnp.float32),
        mesh=mesh,
        scratch_types=[
            pltpu.VMEM((CHUNK,), jnp.int32),
            pltpu.VMEM((CHUNK,), jnp.int32),
            pltpu.VMEM((CHUNK,), jnp.int32),
            pltpu.VMEM((CHUNK,), jnp.int32),
            pltpu.VMEM((CHUNK,), jnp.float32),
            pltpu.VMEM((CHUNK,), jnp.float32),
            pltpu.VMEM((CHUNK, D), jnp.float32),
            pltpu.VMEM((CHUNK, D), jnp.float32),
            pltpu.VMEM_SHARED((N_NODES, D), jnp.float32),
            pltpu.SemaphoreType.DMA,
            pltpu.SemaphoreType.DMA,
        ],
    )
    def spmm_kernel(h_hbm, row_hbm, col_hbm, val_hbm, out_hbm,
                    col_a, col_b, row_a, row_b, val_a, val_b,
                    rows_a, rows_b, acc_sh,
                    sem_a, sem_b):
        cid = lax.axis_index("c")
        sid = lax.axis_index("s")

        @pl.loop(0, RBLK)
        def _(e):
            for k in range(D // L):
                rows_a[e, pl.ds(k * L, L)] = jnp.zeros((L,), jnp.float32)

        @pl.loop(0, RB_T)
        def _(t):
            blk = sid + t * NS

            @pl.when(blk < N_RBLK)
            def _():
                pltpu.sync_copy(rows_a.at[pl.ds(0, RBLK)],
                                acc_sh.at[pl.ds(blk * RBLK, RBLK)])

        plsc.subcore_barrier()
        wid = sid * NC + cid
        NW = NC * NS

        def load_idx(m, cbuf, rbuf, vbuf):
            off = m * CHUNK
            pltpu.sync_copy(col_hbm.at[pl.ds(off, CHUNK)], cbuf)
            pltpu.sync_copy(row_hbm.at[pl.ds(off, CHUNK)], rbuf)
            pltpu.sync_copy(val_hbm.at[pl.ds(off, CHUNK)], vbuf)

        def start_gather(cbuf, buf, sem):
            pltpu.async_copy(h_hbm.at[cbuf], buf, sem)

        def wait_gather(cbuf, buf, sem):
            pltpu.make_async_copy(h_hbm.at[cbuf], buf, sem).wait()

        def scale(buf, vbuf):
            @pl.loop(0, CHUNK // L)
            def _(g):
                for e in range(L):
                    bcast = plsc.load_gather(
                        vbuf, [jnp.full((L,), g * L + e, jnp.int32)])
                    r = g * L + e
                    for k in range(D // L):
                        sl = pl.ds(k * L, L)
                        buf[r, sl] = buf[r, sl] * bcast

        def scatter(buf, rbuf):
            pltpu.sync_copy(buf, acc_sh.at[rbuf], add=True)

        load_idx(wid, col_a, row_a, val_a)
        start_gather(col_a, rows_a, sem_a)

        @pl.loop(0, CPW // 2)
        def _(t):
            m0 = wid + (t * 2) * NW

            load_idx(m0 + NW, col_b, row_b, val_b)
            start_gather(col_b, rows_b, sem_b)

            wait_gather(col_a, rows_a, sem_a)
            scale(rows_a, val_a)
            scatter(rows_a, row_a)

            @pl.when(t + 1 < CPW // 2)
            def _():
                load_idx(m0 + 2 * NW, col_a, row_a, val_a)
                start_gather(col_a, rows_a, sem_a)

            wait_gather(col_b, rows_b, sem_b)
            scale(rows_b, val_b)
            scatter(rows_b, row_b)

        plsc.subcore_barrier()

        @pl.loop(0, RB_T)
        def _(t):
            blk = sid + t * NS

            @pl.when(blk < N_RBLK)
            def _():
                pltpu.sync_copy(
                    acc_sh.at[pl.ds(blk * RBLK, RBLK)],
                    out_hbm.at[cid, pl.ds(blk * RBLK, RBLK)])

    return spmm_kernel(h, row2, col, val)


def kernel(x, edge_index, adj_values, W, b):
    row = edge_index[0].astype(jnp.int32)
    col = edge_index[1].astype(jnp.int32)
    val = adj_values.astype(jnp.float32)

    pad = E_PAD - N_EDGES
    row2 = jnp.pad(row, (0, pad))
    col = jnp.pad(col, (0, pad))
    val = jnp.pad(val, (0, pad))

    h = pl.pallas_call(
        _matmul_body,
        grid=(10,),
        in_specs=[
            pl.BlockSpec((N_NODES // 10, D), lambda i: (i, 0)),
            pl.BlockSpec((D, D), lambda i: (0, 0)),
        ],
        out_specs=pl.BlockSpec((N_NODES // 10, D), lambda i: (i, 0)),
        out_shape=jax.ShapeDtypeStruct((N_NODES, D), jnp.float32),
    )(x, W)

    partials = _sc_spmm(h, row2, col, val)

    b2 = b.reshape(1, D).astype(jnp.float32)
    out = pl.pallas_call(
        _combine_body,
        grid=(10,),
        in_specs=[
            pl.BlockSpec((NC, N_NODES // 10, D), lambda i: (0, i, 0)),
            pl.BlockSpec((1, D), lambda i: (0, 0)),
        ],
        out_specs=pl.BlockSpec((N_NODES // 10, D), lambda i: (i, 0)),
        out_shape=jax.ShapeDtypeStruct((N_NODES, D), jnp.float32),
    )(partials, b2)
    return out


# speedup vs baseline: 1.1556x; 1.0182x over previous
import dataclasses
import functools

import jax
import jax.numpy as jnp
from jax import lax
from jax.experimental import pallas as pl
from jax.experimental.pallas import tpu as pltpu
from jax.experimental.pallas import tpu_sc as plsc

N_NODES = 10000
N_EDGES = 320000
D = 128

NC = 2
NS = 16
L = 16

CHUNK = 128
CPW = 80
N_CHUNKS = NC * NS * CPW
N_SUPER = N_CHUNKS // 2
SPW = CPW // 2
E_PAD = N_CHUNKS * CHUNK
RBLK = 80
N_RBLK = N_NODES // RBLK
RB_T = (N_RBLK + NS - 1) // NS


def _matmul_body(x_ref, w_ref, o_ref):
    o_ref[...] = jnp.dot(x_ref[...], w_ref[...],
                         preferred_element_type=jnp.float32)


def _combine_body(p_ref, b_ref, o_ref):
    o_ref[...] = p_ref[0] + p_ref[1] + b_ref[...]


def _sc_spmm(h, packed):
    mesh = plsc.VectorSubcoreMesh(core_axis_name="c", subcore_axis_name="s")
    cp = pltpu.CompilerParams()
    if "needs_layout_passes" in pltpu.CompilerParams.__dataclass_fields__:
        cp = dataclasses.replace(cp, needs_layout_passes=False)

    @functools.partial(
        pl.kernel,
        compiler_params=cp,
        out_type=jax.ShapeDtypeStruct((NC, N_NODES, D), jnp.float32),
        mesh=mesh,
        scratch_types=[
            pltpu.VMEM((8, CHUNK), jnp.int32),
            pltpu.VMEM((2 * CHUNK, D), jnp.float32),
            pltpu.VMEM_SHARED((N_NODES, D), jnp.float32),
            pltpu.SemaphoreType.DMA,
        ],
    )
    def spmm_kernel(h_hbm, packed_hbm, out_hbm,
                    idx2, rows_v, acc_sh, sem):
        cid = lax.axis_index("c")
        sid = lax.axis_index("s")

        @pl.loop(0, RBLK)
        def _(e):
            for k in range(D // L):
                rows_v[e, pl.ds(k * L, L)] = jnp.zeros((L,), jnp.float32)

        @pl.loop(0, RB_T)
        def _(t):
            blk = sid + t * NS

            @pl.when(blk < N_RBLK)
            def _():
                pltpu.sync_copy(rows_v.at[pl.ds(0, RBLK)],
                                acc_sh.at[pl.ds(blk * RBLK, RBLK)])

        plsc.subcore_barrier()
        wid = sid * NC + cid
        NW = NC * NS

        @pl.loop(0, SPW)
        def _(t):
            m = wid + t * NW
            pltpu.sync_copy(packed_hbm.at[m], idx2)
            pltpu.async_copy(
                h_hbm.at[idx2.at[0]], rows_v.at[pl.ds(0, CHUNK)], sem)
            pltpu.async_copy(
                h_hbm.at[idx2.at[1]], rows_v.at[pl.ds(CHUNK, CHUNK)], sem)
            pltpu.make_async_copy(
                h_hbm.at[idx2.at[0]], rows_v.at[pl.ds(0, CHUNK)], sem).wait()
            pltpu.make_async_copy(
                h_hbm.at[idx2.at[1]], rows_v.at[pl.ds(CHUNK, CHUNK)],
                sem).wait()

            @pl.loop(0, 2 * CHUNK // L)
            def _(g):
                vrow = 4 + g // (CHUNK // L)
                lane0 = (g % (CHUNK // L)) * L
                for e in range(L):
                    bits = plsc.load_gather(
                        idx2, [jnp.full((L,), vrow, jnp.int32),
                               jnp.full((L,), lane0 + e, jnp.int32)])
                    bcast = plsc.bitcast(bits, jnp.float32)
                    r = g * L + e
                    for k in range(D // L):
                        sl = pl.ds(k * L, L)
                        rows_v[r, sl] = rows_v[r, sl] * bcast

            pltpu.sync_copy(rows_v.at[pl.ds(0, CHUNK)],
                            acc_sh.at[idx2.at[2]], add=True)
            pltpu.sync_copy(rows_v.at[pl.ds(CHUNK, CHUNK)],
                            acc_sh.at[idx2.at[3]], add=True)

        plsc.subcore_barrier()

        @pl.loop(0, RB_T)
        def _(t):
            blk = sid + t * NS

            @pl.when(blk < N_RBLK)
            def _():
                pltpu.sync_copy(
                    acc_sh.at[pl.ds(blk * RBLK, RBLK)],
                    out_hbm.at[cid, pl.ds(blk * RBLK, RBLK)])

    return spmm_kernel(h, packed)


def kernel(x, edge_index, adj_values, W, b):
    row = edge_index[0].astype(jnp.int32)
    col = edge_index[1].astype(jnp.int32)
    val = adj_values.astype(jnp.float32)

    pad = E_PAD - N_EDGES
    c3 = jnp.pad(col, (0, pad)).reshape(N_SUPER, 2, CHUNK)
    r3 = jnp.pad(row, (0, pad)).reshape(N_SUPER, 2, CHUNK)
    v3 = jax.lax.bitcast_convert_type(
        jnp.pad(val, (0, pad)), jnp.int32).reshape(N_SUPER, 2, CHUNK)
    z3 = jnp.zeros((N_SUPER, 2, CHUNK), jnp.int32)
    packed = jnp.concatenate([c3, r3, v3, z3], axis=1)

    h = pl.pallas_call(
        _matmul_body,
        grid=(10,),
        in_specs=[
            pl.BlockSpec((N_NODES // 10, D), lambda i: (i, 0)),
            pl.BlockSpec((D, D), lambda i: (0, 0)),
        ],
        out_specs=pl.BlockSpec((N_NODES // 10, D), lambda i: (i, 0)),
        out_shape=jax.ShapeDtypeStruct((N_NODES, D), jnp.float32),
    )(x, W)

    partials = _sc_spmm(h, packed)

    b2 = b.reshape(1, D).astype(jnp.float32)
    out = pl.pallas_call(
        _combine_body,
        grid=(10,),
        in_specs=[
            pl.BlockSpec((NC, N_NODES // 10, D), lambda i: (0, i, 0)),
            pl.BlockSpec((1, D), lambda i: (0, 0)),
        ],
        out_specs=pl.BlockSpec((N_NODES // 10, D), lambda i: (i, 0)),
        out_shape=jax.ShapeDtypeStruct((N_NODES, D), jnp.float32),
    )(partials, b2)
    return out
